# Initial kernel scaffold; baseline (speedup 1.0000x reference)
import jax
import jax.numpy as jnp
from jax.experimental import pallas as pl


def kernel(x, edge_index, W1, b1, W2, b2, fc1_w, fc1_b, fc2_w, fc2_b):
    raise NotImplementedError("write your pallas kernel here")


# speedup vs baseline: 16.5230x; 16.5230x over previous
import functools

import jax
import jax.numpy as jnp
from jax import lax
from jax.experimental import pallas as pl
from jax.experimental.pallas import tpu as pltpu
from jax.experimental.pallas import tpu_sc as plsc

_NC = 2
_NS = 16
_EB = 128
_DEGW = 16
_RB = 128


def _sc_agg(np_total, width, num_chunks, with_table):
  rows_per_tile = np_total // _NS
  mesh = plsc.VectorSubcoreMesh(core_axis_name="c", subcore_axis_name="s")

  scratch = []
  if with_table:
    scratch.append(pltpu.VMEM((_EB,), jnp.int32))
    scratch.append(pltpu.VMEM_SHARED((np_total, width), jnp.float32))
  scratch += [
      pltpu.VMEM((_EB,), jnp.int32),
      pltpu.VMEM((_EB, width), jnp.float32),
      pltpu.VMEM((rows_per_tile, width), jnp.float32),
      pltpu.VMEM_SHARED((np_total, width), jnp.float32),
      pltpu.SemaphoreType.DMA,
  ]

  def body(*refs):
    if with_table:
      (table, src, dst, zrows, out,
       src_v, table_sh, dst_v, rows_v, stage_v, acc_sh, sem) = refs
    else:
      (dst, zrows, orows, out,
       dst_v, rows_v, stage_v, acc_sh, sem) = refs
    cid = lax.axis_index("c")
    sid = lax.axis_index("s")
    row0 = sid * rows_per_tile
    tile_rows = pl.ds(row0, rows_per_tile)

    pltpu.sync_copy(zrows, stage_v)
    pltpu.sync_copy(stage_v, acc_sh.at[tile_rows])
    if with_table:
      pltpu.sync_copy(table.at[tile_rows], stage_v)
      pltpu.sync_copy(stage_v, table_sh.at[tile_rows])
    else:
      pltpu.sync_copy(orows, rows_v)
    plsc.subcore_barrier()

    per_core = _NS * num_chunks * _EB
    tile_base = cid * per_core + sid * (num_chunks * _EB)

    def chunk(k, carry):
      base = tile_base + k * _EB
      pltpu.sync_copy(dst.at[pl.ds(base, _EB)], dst_v)
      if with_table:
        pltpu.sync_copy(src.at[pl.ds(base, _EB)], src_v)
        pltpu.async_copy(table_sh.at[src_v], rows_v, sem).wait()
      pltpu.sync_copy(rows_v, acc_sh.at[dst_v], add=True)
      return carry

    lax.fori_loop(0, num_chunks, chunk, 0)
    plsc.subcore_barrier()

    pltpu.sync_copy(acc_sh.at[tile_rows], stage_v)
    pltpu.sync_copy(stage_v, out.at[cid, tile_rows])

  return pl.kernel(
      body,
      out_type=jax.ShapeDtypeStruct((_NC, np_total, width), jnp.float32),
      mesh=mesh,
      scratch_types=scratch,
      compiler_params=pltpu.CompilerParams(use_tc_tiling_on_sc=False),
  )


def _dinv_block(degp_ref):
  deg = degp_ref[0, :, :1] + degp_ref[1, :, :1] + 1.0
  return lax.rsqrt(deg)


def _tc1_body(degp_ref, x_ref, w_ref, out_ref):
  dinv = _dinv_block(degp_ref)
  h = jnp.dot(x_ref[...], w_ref[...], preferred_element_type=jnp.float32)
  out_ref[...] = h * dinv


def _tc2_body(n_real, degp_ref, aggp_ref, h1s_ref, w_ref, b_ref, out_ref):
  i = pl.program_id(0)
  dinv = _dinv_block(degp_ref)
  a = aggp_ref[0] + aggp_ref[1] + h1s_ref[...]
  hh = jnp.maximum(a * dinv + b_ref[...], 0.0)
  rows = i * _RB + lax.broadcasted_iota(jnp.int32, (_RB, 1), 0)
  hh = jnp.where(rows < n_real, hh, 0.0)
  out_ref[...] = jnp.dot(hh, w_ref[...], preferred_element_type=jnp.float32) * dinv


def _tc3_body(n_real, ng, degp_ref, aggp_ref, h2s_ref, b2_ref,
              fc1w_ref, fc1b_ref, fc2w_ref, fc2b_ref, out_ref, acc_ref):
  i = pl.program_id(0)
  dinv = _dinv_block(degp_ref)
  a = aggp_ref[0] + aggp_ref[1] + h2s_ref[...]
  hh = jnp.maximum(a * dinv + b2_ref[...], 0.0)
  rows = i * _RB + lax.broadcasted_iota(jnp.int32, (_RB, 1), 0)
  hh = jnp.where(rows < n_real, hh, 0.0)
  part = jnp.sum(hh, axis=0, keepdims=True)

  @pl.when(i == 0)
  def _():
    acc_ref[...] = part

  @pl.when(i > 0)
  def _():
    acc_ref[...] = acc_ref[...] + part

  @pl.when(i == ng - 1)
  def _():
    g = acc_ref[...] * (1.0 / n_real)
    g1 = jnp.maximum(
        jnp.dot(g, fc1w_ref[...], preferred_element_type=jnp.float32)
        + fc1b_ref[...], 0.0)
    logits = (jnp.dot(g1, fc2w_ref[...], preferred_element_type=jnp.float32)
              + fc2b_ref[...])
    out_ref[...] = 1.0 / (1.0 + jnp.exp(-logits))


def kernel(x, edge_index, W1, b1, W2, b2, fc1_w, fc1_b, fc2_w, fc2_b):
  n, d = x.shape
  h = W1.shape[1]
  o = fc2_w.shape[1]
  e = edge_index.shape[1]

  np_total = ((n + 1 + _RB - 1) // _RB) * _RB
  rows_per_tile = np_total // _NS
  ng = np_total // _RB
  num_chunks = (e + _NC * _NS * _EB - 1) // (_NC * _NS * _EB)
  ep = _NC * _NS * _EB * num_chunks

  pad = jnp.full((ep - e,), n, dtype=jnp.int32)
  src = jnp.concatenate([edge_index[0].astype(jnp.int32), pad])
  dst = jnp.concatenate([edge_index[1].astype(jnp.int32), pad])
  x_pad = jnp.zeros((np_total, d), jnp.float32).at[:n].set(x)
  zrows_h = jnp.zeros((rows_per_tile, h), jnp.float32)
  zrows_d = jnp.zeros((rows_per_tile, _DEGW), jnp.float32)
  ones_rows = jnp.ones((_EB, _DEGW), jnp.float32)

  deg_fn = _sc_agg(np_total, _DEGW, num_chunks, with_table=False)
  agg_fn = _sc_agg(np_total, h, num_chunks, with_table=True)

  degp = deg_fn(dst, zrows_d, ones_rows)

  h1s = pl.pallas_call(
      _tc1_body,
      grid=(ng,),
      in_specs=[
          pl.BlockSpec((_NC, _RB, _DEGW), lambda i: (0, i, 0)),
          pl.BlockSpec((_RB, d), lambda i: (i, 0)),
          pl.BlockSpec((d, h), lambda i: (0, 0)),
      ],
      out_specs=pl.BlockSpec((_RB, h), lambda i: (i, 0)),
      out_shape=jax.ShapeDtypeStruct((np_total, h), jnp.float32),
  )(degp, x_pad, W1)

  agg1 = agg_fn(h1s, src, dst, zrows_h)

  h2s = pl.pallas_call(
      functools.partial(_tc2_body, n),
      grid=(ng,),
      in_specs=[
          pl.BlockSpec((_NC, _RB, _DEGW), lambda i: (0, i, 0)),
          pl.BlockSpec((_NC, _RB, h), lambda i: (0, i, 0)),
          pl.BlockSpec((_RB, h), lambda i: (i, 0)),
          pl.BlockSpec((h, h), lambda i: (0, 0)),
          pl.BlockSpec((1, h), lambda i: (0, 0)),
      ],
      out_specs=pl.BlockSpec((_RB, h), lambda i: (i, 0)),
      out_shape=jax.ShapeDtypeStruct((np_total, h), jnp.float32),
  )(degp, agg1, h1s, W2, b1.reshape(1, h))

  agg2 = agg_fn(h2s, src, dst, zrows_h)

  out = pl.pallas_call(
      functools.partial(_tc3_body, n, ng),
      grid=(ng,),
      in_specs=[
          pl.BlockSpec((_NC, _RB, _DEGW), lambda i: (0, i, 0)),
          pl.BlockSpec((_NC, _RB, h), lambda i: (0, i, 0)),
          pl.BlockSpec((_RB, h), lambda i: (i, 0)),
          pl.BlockSpec((1, h), lambda i: (0, 0)),
          pl.BlockSpec((h, h), lambda i: (0, 0)),
          pl.BlockSpec((1, h), lambda i: (0, 0)),
          pl.BlockSpec((h, o), lambda i: (0, 0)),
          pl.BlockSpec((1, o), lambda i: (0, 0)),
      ],
      out_specs=pl.BlockSpec((1, o), lambda i: (0, 0)),
      out_shape=jax.ShapeDtypeStruct((1, o), jnp.float32),
      scratch_shapes=[pltpu.VMEM((1, h), jnp.float32)],
  )(degp, agg2, h2s, b2.reshape(1, h), fc1_w, fc1_b.reshape(1, h),
    fc2_w, fc2_b.reshape(1, o))

  return out.reshape(o)
